# baseline (speedup 1.0000x reference)
import functools
import math

import jax
import jax.numpy as jnp
from jax import lax
from jax.experimental import pallas as pl
from jax.experimental.pallas import tpu as pltpu
from jax.experimental.pallas import tpu_sc as plsc

D_EMBED = 128
SCALE = math.sqrt(D_EMBED)

NC = 2
NS = 16
NW = NC * NS

NB = 5
PF = 4


def _scale_body(t_ref, o_ref):
    o_ref[...] = t_ref[...] * SCALE


def _scaled_table(table):
    v, d = table.shape
    blk = 2000
    return pl.pallas_call(
        _scale_body,
        out_shape=jax.ShapeDtypeStruct((v, d), jnp.float32),
        grid=(v // blk,),
        in_specs=[pl.BlockSpec((blk, d), lambda i: (i, 0))],
        out_specs=pl.BlockSpec((blk, d), lambda i: (i, 0)),
    )(table)


def _make_gather(n_groups):
    gpw = n_groups // NW
    assert gpw % NB == 0 and gpw >= 2 * NB
    mesh = plsc.VectorSubcoreMesh(core_axis_name="c", subcore_axis_name="s")

    @functools.partial(
        pl.kernel,
        out_type=jax.ShapeDtypeStruct((n_groups * 128, D_EMBED), jnp.float32),
        mesh=mesh,
        scratch_types=[
            pltpu.VMEM((gpw, 128), jnp.int32),
            pltpu.VMEM((NB, 128, D_EMBED), jnp.float32),
            pltpu.SemaphoreType.DMA((NB,)),
            pltpu.SemaphoreType.DMA((NB,)),
        ],
    )
    def gather(table_hbm, idx_hbm, out_hbm, idx_v, rows_v, semg, semo):
        wid = lax.axis_index("s") * NC + lax.axis_index("c")
        g0 = wid * gpw

        pltpu.sync_copy(idx_hbm.at[pl.ds(g0, gpw)], idx_v)

        def fire_gather(v, b):
            pltpu.async_copy(table_hbm.at[idx_v.at[v]], rows_v.at[b], semg.at[b])

        def wait_gather(b):
            pltpu.make_async_copy(
                table_hbm.at[pl.ds(0, 128)], rows_v.at[b], semg.at[b]
            ).wait()

        def fire_out(v, b):
            pltpu.async_copy(
                rows_v.at[b], out_hbm.at[pl.ds((g0 + v) * 128, 128)], semo.at[b]
            )

        def wait_out(b):
            pltpu.make_async_copy(
                rows_v.at[b], out_hbm.at[pl.ds(0, 128)], semo.at[b]
            ).wait()

        def scale_buf(b):
            def row_body(r, carry):
                for u in range(2):
                    for c in range(8):
                        sl = (2 * r + u, pl.ds(c * 16, 16))
                        rows_v[(b, *sl)] = rows_v[(b, *sl)] * SCALE
                return carry

            lax.fori_loop(0, 64, row_body, 0)

        for v in range(NB):
            fire_gather(v, v)
        for u in range(NB - PF):
            wait_gather(u)
            scale_buf(u)
            fire_out(u, u)

        def body(i, carry):
            for b in range(NB):
                v = i * NB + b
                wait_out(b)
                fire_gather(v, b)
                b2 = (b - PF) % NB
                wait_gather(b2)
                scale_buf(b2)
                fire_out(v - PF, b2)
            return carry

        lax.fori_loop(1, gpw // NB, body, 0)

        for u in range(gpw - PF, gpw):
            b = u % NB
            wait_gather(b)
            scale_buf(b)
            fire_out(u, b)
        for b in range(NB):
            wait_out(b)

    return gather


def kernel(x, table):
    b, t = x.shape
    n = b * t
    idx2d = x.reshape(n // 128, 128).astype(jnp.int32)
    out = _make_gather(n // 128)(table, idx2d)
    return out.reshape(b, t, D_EMBED)


# speedup vs baseline: 1.0085x; 1.0085x over previous
import functools
import math

import jax
import jax.numpy as jnp
from jax import lax
from jax.experimental import pallas as pl
from jax.experimental.pallas import tpu as pltpu
from jax.experimental.pallas import tpu_sc as plsc

D_EMBED = 128
SCALE = math.sqrt(D_EMBED)

NC = 2
NS = 16
NW = NC * NS

NB = 6
PF = 4


def _scale_body(t_ref, o_ref):
    o_ref[...] = t_ref[...] * SCALE


def _scaled_table(table):
    v, d = table.shape
    blk = 2000
    return pl.pallas_call(
        _scale_body,
        out_shape=jax.ShapeDtypeStruct((v, d), jnp.float32),
        grid=(v // blk,),
        in_specs=[pl.BlockSpec((blk, d), lambda i: (i, 0))],
        out_specs=pl.BlockSpec((blk, d), lambda i: (i, 0)),
    )(table)


def _make_gather(n_groups):
    gpw = n_groups // NW
    n_main = gpw // NB
    tail = gpw - n_main * NB
    assert tail < NB and gpw >= 2 * NB
    mesh = plsc.VectorSubcoreMesh(core_axis_name="c", subcore_axis_name="s")

    @functools.partial(
        pl.kernel,
        out_type=jax.ShapeDtypeStruct((n_groups * 128, D_EMBED), jnp.float32),
        mesh=mesh,
        scratch_types=[
            pltpu.VMEM((gpw, 128), jnp.int32),
            pltpu.VMEM((NB, 128, D_EMBED), jnp.float32),
            pltpu.SemaphoreType.DMA((NB,)),
            pltpu.SemaphoreType.DMA((NB,)),
        ],
    )
    def gather(table_hbm, idx_hbm, out_hbm, idx_v, rows_v, semg, semo):
        wid = lax.axis_index("s") * NC + lax.axis_index("c")
        g0 = wid * gpw

        pltpu.sync_copy(idx_hbm.at[pl.ds(g0, gpw)], idx_v)

        def fire_gather(v, b):
            pltpu.async_copy(table_hbm.at[idx_v.at[v]], rows_v.at[b], semg.at[b])

        def wait_gather(b):
            pltpu.make_async_copy(
                table_hbm.at[pl.ds(0, 128)], rows_v.at[b], semg.at[b]
            ).wait()

        def fire_out(v, b):
            pltpu.async_copy(
                rows_v.at[b], out_hbm.at[pl.ds((g0 + v) * 128, 128)], semo.at[b]
            )

        def wait_out(b):
            pltpu.make_async_copy(
                rows_v.at[b], out_hbm.at[pl.ds(0, 128)], semo.at[b]
            ).wait()

        def scale_buf(b):
            def row_body(r, carry):
                for u in range(2):
                    for c in range(8):
                        sl = (2 * r + u, pl.ds(c * 16, 16))
                        rows_v[(b, *sl)] = rows_v[(b, *sl)] * SCALE
                return carry

            lax.fori_loop(0, 64, row_body, 0)

        for v in range(NB):
            fire_gather(v, v)
        for u in range(NB - PF):
            wait_gather(u)
            scale_buf(u)
            fire_out(u, u)

        def body(i, carry):
            for b in range(NB):
                v = i * NB + b
                wait_out(b)
                fire_gather(v, b)
                b2 = (b - PF) % NB
                wait_gather(b2)
                scale_buf(b2)
                fire_out(v - PF, b2)
            return carry

        lax.fori_loop(1, n_main, body, 0)

        for v in range(n_main * NB, gpw):
            b = v % NB
            wait_out(b)
            fire_gather(v, b)
            b2 = (b - PF) % NB
            wait_gather(b2)
            scale_buf(b2)
            fire_out(v - PF, b2)

        for u in range(gpw - PF, gpw):
            b = u % NB
            wait_gather(b)
            scale_buf(b)
            fire_out(u, b)
        for b in range(NB):
            wait_out(b)

    return gather


def kernel(x, table):
    b, t = x.shape
    n = b * t
    idx2d = x.reshape(n // 128, 128).astype(jnp.int32)
    out = _make_gather(n // 128)(table, idx2d)
    return out.reshape(b, t, D_EMBED)
